# baseline (speedup 1.0000x reference)
import functools

import jax
import jax.numpy as jnp
import numpy as np
from jax import lax
from jax.experimental import pallas as pl
from jax.experimental.pallas import tpu as pltpu
from jax.experimental.pallas import tpu_sc as plsc

_EMBED = 128
_TIME = 100
_EDGE = 16
_QDIM = _EMBED + _TIME
_DH = _QDIM // 2
_NN = 100000
_B0 = 2048
_K = 16
_B1 = _B0 * _K
_NU = _B0 + _B1

_NC = 2
_NS = 16
_NW = _NC * _NS
_L = 16
_C = 128


def _wid():
    return lax.axis_index("s") * _NC + lax.axis_index("c")


def _sc_elem_gather(table, idx):
    n = idx.shape[0]
    t = table.shape[0]
    per_w = n // _NW
    nv = per_w // _L
    mesh = plsc.VectorSubcoreMesh(core_axis_name="c", subcore_axis_name="s")

    @functools.partial(
        pl.kernel,
        mesh=mesh,
        compiler_params=pltpu.CompilerParams(needs_layout_passes=False),
        out_type=jax.ShapeDtypeStruct((n,), jnp.int32),
        scratch_types=[
            pltpu.VMEM((t,), jnp.int32),
            pltpu.VMEM((per_w,), jnp.int32),
            pltpu.VMEM((per_w,), jnp.int32),
        ],
    )
    def k(table_hbm, idx_hbm, out_hbm, table_v, idx_v, out_v):
        w = _wid()
        base = w * per_w
        pltpu.sync_copy(table_hbm, table_v)
        pltpu.sync_copy(idx_hbm.at[pl.ds(base, per_w)], idx_v)

        def body(i, carry):
            iv = idx_v[pl.ds(i * _L, _L)]
            out_v[pl.ds(i * _L, _L)] = plsc.load_gather(table_v, [iv])
            return carry

        lax.fori_loop(0, nv, body, 0)
        pltpu.sync_copy(out_v, out_hbm.at[pl.ds(base, per_w)])

    return k(table, idx)


_NB = 4


def _sc_row_gather(table, idx):
    n = idx.shape[0]
    d = table.shape[1]
    per_w = n // _NW
    chunks = per_w // _C
    ngroups = chunks // _NB
    mesh = plsc.VectorSubcoreMesh(core_axis_name="c", subcore_axis_name="s")

    @functools.partial(
        pl.kernel,
        mesh=mesh,
        compiler_params=pltpu.CompilerParams(needs_layout_passes=False),
        out_type=jax.ShapeDtypeStruct((n, d), jnp.float32),
        scratch_types=[
            pltpu.VMEM((per_w,), jnp.int32),
        ] + [pltpu.VMEM((_C, d), jnp.float32)] * _NB
          + [pltpu.SemaphoreType.DMA] * (2 * _NB),
    )
    def k(table_hbm, idx_hbm, out_hbm, idx_v, *rest):
        bufs = rest[:_NB]
        gs = rest[_NB:2 * _NB]
        ws = rest[2 * _NB:]
        w = _wid()
        base = w * per_w
        pltpu.sync_copy(idx_hbm.at[pl.ds(base, per_w)], idx_v)

        def g_start(c, j):
            pltpu.async_copy(
                table_hbm.at[idx_v.at[pl.ds(c * _C, _C)]], bufs[j], gs[j])

        def g_wait(j):
            pltpu.make_async_copy(
                table_hbm.at[idx_v.at[pl.ds(0, _C)]], bufs[j], gs[j]).wait()

        def w_start(c, j):
            pltpu.async_copy(bufs[j], out_hbm.at[pl.ds(base + c * _C, _C)],
                             ws[j])

        def w_wait(j):
            pltpu.make_async_copy(bufs[j], out_hbm.at[pl.ds(base, _C)],
                                  ws[j]).wait()

        for j in range(_NB):
            g_start(j, j)

        def body(g, carry):
            c0 = g * _NB
            for j in range(_NB):
                g_wait(j)
                w_start(c0 + j, j)
            for j in range(_NB):
                w_wait(j)
                g_start(c0 + _NB + j, j)
            return carry

        lax.fori_loop(0, ngroups - 1, body, 0)
        c0 = (ngroups - 1) * _NB
        for j in range(_NB):
            g_wait(j)
            w_start(c0 + j, j)
        for j in range(_NB):
            w_wait(j)

    return k(table, idx)


def _layer_block(combine, bn, *refs):
    if combine:
        (nf_r, nbrf_r, ef_r, dt_r, mk_r, nbrz_r, selz_r, sels_r,
         wqa_r, qc_r, wka_r, wkb_r, wkc_r, bk_r,
         wva_r, wvb_r, wvc_r, bv_r, wo_r, w1a_r, w1b_r, b1_r, w2_r, b2_r,
         tw_r, tb_r, out_r) = refs
    else:
        (nf_r, nbrf_r, ef_r, dt_r, mk_r,
         wqa_r, qc_r, wka_r, wkb_r, wkc_r, bk_r,
         wva_r, wvb_r, wvc_r, bv_r, wo_r, w1a_r, w1b_r, b1_r, w2_r, b2_r,
         tw_r, tb_r, out_r) = refs

    nf = nf_r[...]
    nbr = nbrf_r[...]
    if combine:
        nbr = selz_r[...] * nbrz_r[...] + sels_r[...] * nbr
    tf = jnp.cos(dt_r[...] * tw_r[...] + tb_r[...])

    bf = jnp.bfloat16
    kmat = (jnp.dot(nbr, wka_r[...], preferred_element_type=jnp.float32)
            + jnp.dot(ef_r[...], wkb_r[...], preferred_element_type=jnp.float32)
            + jnp.dot(tf, wkc_r[...], preferred_element_type=jnp.float32)
            + bk_r[...])
    vmat = (jnp.dot(nbr.astype(bf), wva_r[...].astype(bf), preferred_element_type=jnp.float32)
            + jnp.dot(ef_r[...].astype(bf), wvb_r[...].astype(bf), preferred_element_type=jnp.float32)
            + jnp.dot(tf.astype(bf), wvc_r[...].astype(bf), preferred_element_type=jnp.float32)
            + bv_r[...])
    q = jnp.dot(nf, wqa_r[...], preferred_element_type=jnp.float32) + qc_r[...]

    k3 = kmat.reshape(bn, _K, _QDIM)
    v3 = vmat.reshape(bn, _K, _QDIM)
    prod = q[:, None, :] * k3
    didx = lax.broadcasted_iota(jnp.int32, (1, 1, _QDIM), 2)
    head0 = didx < _DH
    s_all = jnp.sum(prod, axis=2)
    s0 = jnp.sum(jnp.where(head0, prod, 0.0), axis=2)
    s1 = s_all - s0
    scale = jnp.float32(1.0 / np.sqrt(_DH))
    m = mk_r[...] > 0.0
    s0 = jnp.where(m, s0 * scale, jnp.float32(-1e30))
    s1 = jnp.where(m, s1 * scale, jnp.float32(-1e30))
    e0 = jnp.exp(s0 - jnp.max(s0, axis=1, keepdims=True))
    e1 = jnp.exp(s1 - jnp.max(s1, axis=1, keepdims=True))
    a0 = e0 / jnp.sum(e0, axis=1, keepdims=True)
    a1 = e1 / jnp.sum(e1, axis=1, keepdims=True)
    attn = jnp.where(head0, a0[:, :, None], a1[:, :, None])
    outf = jnp.sum(attn * v3, axis=1)

    h = jnp.dot(outf, wo_r[...], preferred_element_type=jnp.float32)
    x = jnp.maximum(
        jnp.dot(h, w1a_r[...], preferred_element_type=jnp.float32)
        + jnp.dot(nf, w1b_r[...], preferred_element_type=jnp.float32)
        + b1_r[...], 0.0)
    out_r[...] = jnp.dot(x, w2_r[...], preferred_element_type=jnp.float32) + b2_r[...]


def _tc_layer(nf, nbrf, ef, dt, maskf, nbrz, selz, sels, weights, combine):
    n = nf.shape[0]
    bn = 256
    nb = n // bn
    bnk = bn * _K

    def rep(shape):
        return pl.BlockSpec(shape, lambda i: (0, 0))

    in_specs = [
        pl.BlockSpec((bn, _EMBED), lambda i: (i, 0)),
        pl.BlockSpec((bnk, _EMBED), lambda i: (i, 0)),
        pl.BlockSpec((bnk, _EDGE), lambda i: (i, 0)),
        pl.BlockSpec((bnk, 1), lambda i: (i, 0)),
        pl.BlockSpec((bn, _K), lambda i: (i, 0)),
    ]
    args = [nf, nbrf, ef, dt, maskf]
    if combine:
        in_specs += [
            pl.BlockSpec((bnk, _EMBED), lambda i: (i, 0)),
            pl.BlockSpec((bnk, 1), lambda i: (i, 0)),
            pl.BlockSpec((bnk, 1), lambda i: (i, 0)),
        ]
        args += [nbrz, selz, sels]
    in_specs += [rep(w.shape) for w in weights]
    args += list(weights)

    return pl.pallas_call(
        functools.partial(_layer_block, combine, bn),
        grid=(nb,),
        in_specs=in_specs,
        out_specs=pl.BlockSpec((bn, _EMBED), lambda i: (i, 0)),
        out_shape=jax.ShapeDtypeStruct((n, _EMBED), jnp.float32),
    )(*args)


def _prep_weights(Wq, bq, Wk, bk, Wv, bv, Wo, bo, W1, b1, W2, b2, tw, tb):
    qc = (jnp.cos(tb)[None, :] @ Wq[:, _EMBED:].T + bq[None, :])
    wqa = Wq[:, :_EMBED].T
    wka = Wk[:, :_EMBED].T
    wkb = Wk[:, _EMBED:_EMBED + _EDGE].T
    wkc = Wk[:, _EMBED + _EDGE:].T
    wva = Wv[:, :_EMBED].T
    wvb = Wv[:, _EMBED:_EMBED + _EDGE].T
    wvc = Wv[:, _EMBED + _EDGE:].T
    wo = Wo.T
    w1a = W1[:, :_QDIM].T
    w1b = W1[:, _QDIM:].T
    b1e = (b1 + bo @ W1[:, :_QDIM].T)[None, :]
    w2 = W2.T
    return (wqa, qc, wka, wkb, wkc, bk[None, :], wva, wvb, wvc, bv[None, :],
            wo, w1a, w1b, b1e, w2, b2[None, :], tw[None, :], tb[None, :])


def kernel(static_node_feat, nids0, nids1, nbr_nids0, nbr_nids1, nbr_mask0,
           nbr_mask1, times0, times1, nbr_times0, nbr_times1, nbr_feats0,
           nbr_feats1, g2l, tw, tb,
           Wq0, bq0, Wk0, bk0, Wv0, bv0, Wo0, bo0, W10, b10, W20, b20,
           Wq1, bq1, Wk1, bk1, Wv1, bv1, Wo1, bo1, W11, b11, W21, b21):
    f32 = jnp.float32
    i32 = jnp.int32

    remap_idx = jnp.concatenate(
        [nids1, nbr_nids0.reshape(-1), nids0])
    remap = _sc_elem_gather(g2l, remap_idx)
    l1 = remap[:_B1]
    lnbr0 = remap[_B1:_B1 + _B1]
    l0 = remap[_B1 + _B1:]

    feat_idx1 = jnp.concatenate([
        nids1, nbr_nids1.reshape(-1),
        jnp.arange(16384, dtype=i32),
    ])
    rows1 = _sc_row_gather(static_node_feat, feat_idx1)
    nf1 = rows1[:_B1]
    nbrf1 = rows1[_B1:_B1 + _B1 * _K]
    feat_idx0 = jnp.concatenate([
        nids0, nbr_nids0.reshape(-1),
        jnp.arange(14336, dtype=i32),
    ])
    rows0 = _sc_row_gather(static_node_feat, feat_idx0)
    nf0 = rows0[:_B0]
    nbrf0 = rows0[_B0:_B0 + _B0 * _K]

    p1 = _prep_weights(Wq1, bq1, Wk1, bk1, Wv1, bv1, Wo1, bo1, W11, b11,
                       W21, b21, tw, tb)
    m1f = nbr_mask1.astype(f32)
    dt1 = jnp.where(nbr_mask1, times1[:, None] - nbr_times1, 0.0)
    out1 = _tc_layer(nf1, nbrf1, nbr_feats1.reshape(_B1 * _K, _EDGE),
                     dt1.reshape(_B1 * _K, 1), m1f, None, None, None,
                     p1, combine=False)

    win1 = jnp.full((_NU,), -1, i32).at[l1].max(jnp.arange(_B1, dtype=i32))
    win0 = jnp.full((_NU,), -1, i32).at[l0].max(jnp.arange(_B0, dtype=i32))

    w = _sc_elem_gather(win1, lnbr0)
    valid = w >= 0
    zidx = jnp.where(valid, w, jnp.arange(_B1, dtype=i32))
    nbrz = _sc_row_gather(out1, zidx)
    m0flat = nbr_mask0.reshape(-1)
    selz = (m0flat & valid).astype(f32)[:, None]
    sels = (~m0flat).astype(f32)[:, None]

    p0 = _prep_weights(Wq0, bq0, Wk0, bk0, Wv0, bv0, Wo0, bo0, W10, b10,
                       W20, b20, tw, tb)
    m0f = nbr_mask0.astype(f32)
    dt0 = jnp.where(nbr_mask0, times0[:, None] - nbr_times0, 0.0)
    out0 = _tc_layer(nf0, nbrf0, nbr_feats0.reshape(_B0 * _K, _EDGE),
                     dt0.reshape(_B0 * _K, 1), m0f, nbrz, selz, sels,
                     p0, combine=True)

    table = jnp.concatenate(
        [out1, out0, jnp.zeros((32, _EMBED), f32)], axis=0)
    u = jnp.arange(_NU, dtype=i32)
    winf = jnp.where(win0 >= 0, _B1 + win0,
                     jnp.where(win1 >= 0, win1, _B1 + _B0 + (u % 32)))
    pad = _B1 + _B0 + (jnp.arange(49152 - _NU, dtype=i32) % 32)
    zrows = _sc_row_gather(table, jnp.concatenate([winf, pad]))
    return zrows[:_NU]


# speedup vs baseline: 1.3716x; 1.3716x over previous
import functools

import jax
import jax.numpy as jnp
import numpy as np
from jax import lax
from jax.experimental import pallas as pl
from jax.experimental.pallas import tpu as pltpu
from jax.experimental.pallas import tpu_sc as plsc

_EMBED = 128
_TIME = 100
_EDGE = 16
_QDIM = _EMBED + _TIME
_DH = _QDIM // 2
_NN = 100000
_B0 = 2048
_K = 16
_B1 = _B0 * _K
_NU = _B0 + _B1

_NC = 2
_NS = 16
_NW = _NC * _NS
_L = 16
_C = 128


def _wid():
    return lax.axis_index("s") * _NC + lax.axis_index("c")


def _sc_elem_gather(table, idx):
    n = idx.shape[0]
    t = table.shape[0]
    per_w = n // _NW
    nv = per_w // _L
    mesh = plsc.VectorSubcoreMesh(core_axis_name="c", subcore_axis_name="s")

    @functools.partial(
        pl.kernel,
        mesh=mesh,
        compiler_params=pltpu.CompilerParams(needs_layout_passes=False),
        out_type=jax.ShapeDtypeStruct((n,), jnp.int32),
        scratch_types=[
            pltpu.VMEM((t,), jnp.int32),
            pltpu.VMEM((per_w,), jnp.int32),
            pltpu.VMEM((per_w,), jnp.int32),
        ],
    )
    def k(table_hbm, idx_hbm, out_hbm, table_v, idx_v, out_v):
        w = _wid()
        base = w * per_w
        pltpu.sync_copy(table_hbm, table_v)
        pltpu.sync_copy(idx_hbm.at[pl.ds(base, per_w)], idx_v)

        def body(i, carry):
            iv = idx_v[pl.ds(i * _L, _L)]
            out_v[pl.ds(i * _L, _L)] = plsc.load_gather(table_v, [iv])
            return carry

        lax.fori_loop(0, nv, body, 0)
        pltpu.sync_copy(out_v, out_hbm.at[pl.ds(base, per_w)])

    return k(table, idx)


_NB = 4


def _sc_row_gather(table, idx):
    n = idx.shape[0]
    d = table.shape[1]
    per_w = n // _NW
    chunks = per_w // _C
    ngroups = chunks // _NB
    mesh = plsc.VectorSubcoreMesh(core_axis_name="c", subcore_axis_name="s")

    @functools.partial(
        pl.kernel,
        mesh=mesh,
        compiler_params=pltpu.CompilerParams(needs_layout_passes=False),
        out_type=jax.ShapeDtypeStruct((n, d), jnp.float32),
        scratch_types=[
            pltpu.VMEM((per_w,), jnp.int32),
        ] + [pltpu.VMEM((_C, d), jnp.float32)] * _NB
          + [pltpu.SemaphoreType.DMA] * (2 * _NB),
    )
    def k(table_hbm, idx_hbm, out_hbm, idx_v, *rest):
        bufs = rest[:_NB]
        gs = rest[_NB:2 * _NB]
        ws = rest[2 * _NB:]
        w = _wid()
        base = w * per_w
        pltpu.sync_copy(idx_hbm.at[pl.ds(base, per_w)], idx_v)

        def g_start(c, j):
            pltpu.async_copy(
                table_hbm.at[idx_v.at[pl.ds(c * _C, _C)]], bufs[j], gs[j])

        def g_wait(j):
            pltpu.make_async_copy(
                table_hbm.at[idx_v.at[pl.ds(0, _C)]], bufs[j], gs[j]).wait()

        def w_start(c, j):
            pltpu.async_copy(bufs[j], out_hbm.at[pl.ds(base + c * _C, _C)],
                             ws[j])

        def w_wait(j):
            pltpu.make_async_copy(bufs[j], out_hbm.at[pl.ds(base, _C)],
                                  ws[j]).wait()

        for j in range(_NB):
            g_start(j, j)

        def body(g, carry):
            c0 = g * _NB
            for j in range(_NB):
                g_wait(j)
                w_start(c0 + j, j)
            for j in range(_NB):
                w_wait(j)
                g_start(c0 + _NB + j, j)
            return carry

        lax.fori_loop(0, ngroups - 1, body, 0)
        c0 = (ngroups - 1) * _NB
        for j in range(_NB):
            g_wait(j)
            w_start(c0 + j, j)
        for j in range(_NB):
            w_wait(j)

    return k(table, idx)


_COS_COEF = (0.9999999889445765, -19.739204466576158, 64.93911592834692,
             -85.45011342750767, 60.16742979194585, -25.966884612550658,
             6.527705962902734)


def _fast_cos(x):
    y = x * jnp.float32(1.0 / (2.0 * np.pi))
    big = jnp.float32(12582912.0)
    r = (y + big) - big
    y = y - r
    z = y * y
    acc = jnp.float32(_COS_COEF[-1])
    for c in _COS_COEF[-2::-1]:
        acc = acc * z + jnp.float32(c)
    return acc


def _layer_block(combine, bn, *refs):
    if combine:
        (nf_r, nbrf_r, ef_r, dt_r, mk_r, nbrz_r, selz_r, sels_r,
         wqa_r, qc_r, wka_r, wkb_r, wkc_r, bk_r,
         wva_r, wvb_r, wvc_r, bv_r, wo_r, w1a_r, w1b_r, b1_r, w2_r, b2_r,
         tw_r, tb_r, out_r) = refs
    else:
        (nf_r, nbrf_r, ef_r, dt_r, mk_r,
         wqa_r, qc_r, wka_r, wkb_r, wkc_r, bk_r,
         wva_r, wvb_r, wvc_r, bv_r, wo_r, w1a_r, w1b_r, b1_r, w2_r, b2_r,
         tw_r, tb_r, out_r) = refs

    nf = nf_r[...]
    nbr = nbrf_r[...]
    if combine:
        nbr = selz_r[...] * nbrz_r[...] + sels_r[...] * nbr
    tf = _fast_cos(dt_r[...] * tw_r[...] + tb_r[...])

    kmat = (jnp.dot(nbr, wka_r[...], preferred_element_type=jnp.float32)
            + jnp.dot(ef_r[...], wkb_r[...], preferred_element_type=jnp.float32)
            + jnp.dot(tf, wkc_r[...], preferred_element_type=jnp.float32)
            + bk_r[...])
    vmat = (jnp.dot(nbr, wva_r[...], preferred_element_type=jnp.float32)
            + jnp.dot(ef_r[...], wvb_r[...], preferred_element_type=jnp.float32)
            + jnp.dot(tf, wvc_r[...], preferred_element_type=jnp.float32)
            + bv_r[...])
    q = jnp.dot(nf, wqa_r[...], preferred_element_type=jnp.float32) + qc_r[...]

    k3 = kmat.reshape(bn, _K, _QDIM)
    v3 = vmat.reshape(bn, _K, _QDIM)
    prod = q[:, None, :] * k3
    didx = lax.broadcasted_iota(jnp.int32, (1, 1, _QDIM), 2)
    head0 = didx < _DH
    s_all = jnp.sum(prod, axis=2)
    s0 = jnp.sum(jnp.where(head0, prod, 0.0), axis=2)
    s1 = s_all - s0
    scale = jnp.float32(1.0 / np.sqrt(_DH))
    m = mk_r[...] > 0.0
    s0 = jnp.where(m, s0 * scale, jnp.float32(-1e30))
    s1 = jnp.where(m, s1 * scale, jnp.float32(-1e30))
    e0 = jnp.exp(s0 - jnp.max(s0, axis=1, keepdims=True))
    e1 = jnp.exp(s1 - jnp.max(s1, axis=1, keepdims=True))
    a0 = e0 / jnp.sum(e0, axis=1, keepdims=True)
    a1 = e1 / jnp.sum(e1, axis=1, keepdims=True)
    attn = jnp.where(head0, a0[:, :, None], a1[:, :, None])
    outf = jnp.sum(attn * v3, axis=1)

    h = jnp.dot(outf, wo_r[...], preferred_element_type=jnp.float32)
    x = jnp.maximum(
        jnp.dot(h, w1a_r[...], preferred_element_type=jnp.float32)
        + jnp.dot(nf, w1b_r[...], preferred_element_type=jnp.float32)
        + b1_r[...], 0.0)
    out_r[...] = jnp.dot(x, w2_r[...], preferred_element_type=jnp.float32) + b2_r[...]


def _tc_layer(nf, nbrf, ef, dt, maskf, nbrz, selz, sels, weights, combine):
    n = nf.shape[0]
    bn = 256
    nb = n // bn
    bnk = bn * _K

    def rep(shape):
        return pl.BlockSpec(shape, lambda i: (0, 0))

    in_specs = [
        pl.BlockSpec((bn, _EMBED), lambda i: (i, 0)),
        pl.BlockSpec((bnk, _EMBED), lambda i: (i, 0)),
        pl.BlockSpec((bnk, _EDGE), lambda i: (i, 0)),
        pl.BlockSpec((bnk, 1), lambda i: (i, 0)),
        pl.BlockSpec((bn, _K), lambda i: (i, 0)),
    ]
    args = [nf, nbrf, ef, dt, maskf]
    if combine:
        in_specs += [
            pl.BlockSpec((bnk, _EMBED), lambda i: (i, 0)),
            pl.BlockSpec((bnk, 1), lambda i: (i, 0)),
            pl.BlockSpec((bnk, 1), lambda i: (i, 0)),
        ]
        args += [nbrz, selz, sels]
    in_specs += [rep(w.shape) for w in weights]
    args += list(weights)

    return pl.pallas_call(
        functools.partial(_layer_block, combine, bn),
        grid=(nb,),
        in_specs=in_specs,
        out_specs=pl.BlockSpec((bn, _EMBED), lambda i: (i, 0)),
        out_shape=jax.ShapeDtypeStruct((n, _EMBED), jnp.float32),
    )(*args)


def _prep_weights(Wq, bq, Wk, bk, Wv, bv, Wo, bo, W1, b1, W2, b2, tw, tb):
    qc = (jnp.cos(tb)[None, :] @ Wq[:, _EMBED:].T + bq[None, :])
    wqa = Wq[:, :_EMBED].T
    wka = Wk[:, :_EMBED].T
    wkb = Wk[:, _EMBED:_EMBED + _EDGE].T
    wkc = Wk[:, _EMBED + _EDGE:].T
    wva = Wv[:, :_EMBED].T
    wvb = Wv[:, _EMBED:_EMBED + _EDGE].T
    wvc = Wv[:, _EMBED + _EDGE:].T
    wo = Wo.T
    w1a = W1[:, :_QDIM].T
    w1b = W1[:, _QDIM:].T
    b1e = (b1 + bo @ W1[:, :_QDIM].T)[None, :]
    w2 = W2.T
    return (wqa, qc, wka, wkb, wkc, bk[None, :], wva, wvb, wvc, bv[None, :],
            wo, w1a, w1b, b1e, w2, b2[None, :], tw[None, :], tb[None, :])


def kernel(static_node_feat, nids0, nids1, nbr_nids0, nbr_nids1, nbr_mask0,
           nbr_mask1, times0, times1, nbr_times0, nbr_times1, nbr_feats0,
           nbr_feats1, g2l, tw, tb,
           Wq0, bq0, Wk0, bk0, Wv0, bv0, Wo0, bo0, W10, b10, W20, b20,
           Wq1, bq1, Wk1, bk1, Wv1, bv1, Wo1, bo1, W11, b11, W21, b21):
    f32 = jnp.float32
    i32 = jnp.int32

    remap_idx = jnp.concatenate(
        [nids1, nbr_nids0.reshape(-1), nids0])
    remap = _sc_elem_gather(g2l, remap_idx)
    l1 = remap[:_B1]
    lnbr0 = remap[_B1:_B1 + _B1]
    l0 = remap[_B1 + _B1:]

    feat_idx1 = jnp.concatenate([
        nids1, nbr_nids1.reshape(-1),
        jnp.arange(16384, dtype=i32),
    ])
    rows1 = _sc_row_gather(static_node_feat, feat_idx1)
    nf1 = rows1[:_B1]
    nbrf1 = rows1[_B1:_B1 + _B1 * _K]
    feat_idx0 = jnp.concatenate([
        nids0, nbr_nids0.reshape(-1),
        jnp.arange(14336, dtype=i32),
    ])
    rows0 = _sc_row_gather(static_node_feat, feat_idx0)
    nf0 = rows0[:_B0]
    nbrf0 = rows0[_B0:_B0 + _B0 * _K]

    p1 = _prep_weights(Wq1, bq1, Wk1, bk1, Wv1, bv1, Wo1, bo1, W11, b11,
                       W21, b21, tw, tb)
    m1f = nbr_mask1.astype(f32)
    dt1 = jnp.where(nbr_mask1, times1[:, None] - nbr_times1, 0.0)
    out1 = _tc_layer(nf1, nbrf1, nbr_feats1.reshape(_B1 * _K, _EDGE),
                     dt1.reshape(_B1 * _K, 1), m1f, None, None, None,
                     p1, combine=False)

    win1 = jnp.full((_NU,), -1, i32).at[l1].max(jnp.arange(_B1, dtype=i32))
    win0 = jnp.full((_NU,), -1, i32).at[l0].max(jnp.arange(_B0, dtype=i32))

    w = _sc_elem_gather(win1, lnbr0)
    valid = w >= 0
    zidx = jnp.where(valid, w, jnp.arange(_B1, dtype=i32))
    nbrz = _sc_row_gather(out1, zidx)
    m0flat = nbr_mask0.reshape(-1)
    selz = (m0flat & valid).astype(f32)[:, None]
    sels = (~m0flat).astype(f32)[:, None]

    p0 = _prep_weights(Wq0, bq0, Wk0, bk0, Wv0, bv0, Wo0, bo0, W10, b10,
                       W20, b20, tw, tb)
    m0f = nbr_mask0.astype(f32)
    dt0 = jnp.where(nbr_mask0, times0[:, None] - nbr_times0, 0.0)
    out0 = _tc_layer(nf0, nbrf0, nbr_feats0.reshape(_B0 * _K, _EDGE),
                     dt0.reshape(_B0 * _K, 1), m0f, nbrz, selz, sels,
                     p0, combine=True)

    table = jnp.concatenate(
        [out1, out0, jnp.zeros((32, _EMBED), f32)], axis=0)
    u = jnp.arange(_NU, dtype=i32)
    winf = jnp.where(win0 >= 0, _B1 + win0,
                     jnp.where(win1 >= 0, win1, _B1 + _B0 + (u % 32)))
    pad = _B1 + _B0 + (jnp.arange(49152 - _NU, dtype=i32) % 32)
    zrows = _sc_row_gather(table, jnp.concatenate([winf, pad]))
    return zrows[:_NU]


# speedup vs baseline: 1.4565x; 1.0619x over previous
import functools

import jax
import jax.numpy as jnp
import numpy as np
from jax import lax
from jax.experimental import pallas as pl
from jax.experimental.pallas import tpu as pltpu
from jax.experimental.pallas import tpu_sc as plsc

_EMBED = 128
_TIME = 100
_EDGE = 16
_QDIM = _EMBED + _TIME
_DH = _QDIM // 2
_NN = 100000
_B0 = 2048
_K = 16
_B1 = _B0 * _K
_NU = _B0 + _B1

_NC = 2
_NS = 16
_NW = _NC * _NS
_L = 16
_C = 128


def _wid():
    return lax.axis_index("s") * _NC + lax.axis_index("c")


def _sc_elem_gather(table, idx):
    n = idx.shape[0]
    t = table.shape[0]
    per_w = n // _NW
    nv = per_w // _L
    mesh = plsc.VectorSubcoreMesh(core_axis_name="c", subcore_axis_name="s")

    @functools.partial(
        pl.kernel,
        mesh=mesh,
        compiler_params=pltpu.CompilerParams(needs_layout_passes=False),
        out_type=jax.ShapeDtypeStruct((n,), jnp.int32),
        scratch_types=[
            pltpu.VMEM((t,), jnp.int32),
            pltpu.VMEM((per_w,), jnp.int32),
            pltpu.VMEM((per_w,), jnp.int32),
        ],
    )
    def k(table_hbm, idx_hbm, out_hbm, table_v, idx_v, out_v):
        w = _wid()
        base = w * per_w
        pltpu.sync_copy(table_hbm, table_v)
        pltpu.sync_copy(idx_hbm.at[pl.ds(base, per_w)], idx_v)

        def body(i, carry):
            iv = idx_v[pl.ds(i * _L, _L)]
            out_v[pl.ds(i * _L, _L)] = plsc.load_gather(table_v, [iv])
            return carry

        lax.fori_loop(0, nv, body, 0)
        pltpu.sync_copy(out_v, out_hbm.at[pl.ds(base, per_w)])

    return k(table, idx)


_NB = 4


def _sc_row_gather(table, idx):
    n = idx.shape[0]
    d = table.shape[1]
    per_w = n // _NW
    chunks = per_w // _C
    ngroups = chunks // _NB
    mesh = plsc.VectorSubcoreMesh(core_axis_name="c", subcore_axis_name="s")

    @functools.partial(
        pl.kernel,
        mesh=mesh,
        compiler_params=pltpu.CompilerParams(needs_layout_passes=False),
        out_type=jax.ShapeDtypeStruct((n, d), jnp.float32),
        scratch_types=[
            pltpu.VMEM((per_w,), jnp.int32),
        ] + [pltpu.VMEM((_C, d), jnp.float32)] * _NB
          + [pltpu.SemaphoreType.DMA] * (2 * _NB),
    )
    def k(table_hbm, idx_hbm, out_hbm, idx_v, *rest):
        bufs = rest[:_NB]
        gs = rest[_NB:2 * _NB]
        ws = rest[2 * _NB:]
        w = _wid()
        base = w * per_w
        pltpu.sync_copy(idx_hbm.at[pl.ds(base, per_w)], idx_v)

        def g_start(c, j):
            pltpu.async_copy(
                table_hbm.at[idx_v.at[pl.ds(c * _C, _C)]], bufs[j], gs[j])

        def g_wait(j):
            pltpu.make_async_copy(
                table_hbm.at[idx_v.at[pl.ds(0, _C)]], bufs[j], gs[j]).wait()

        def w_start(c, j):
            pltpu.async_copy(bufs[j], out_hbm.at[pl.ds(base + c * _C, _C)],
                             ws[j])

        def w_wait(j):
            pltpu.make_async_copy(bufs[j], out_hbm.at[pl.ds(base, _C)],
                                  ws[j]).wait()

        for j in range(_NB):
            g_start(j, j)

        def body(g, carry):
            c0 = g * _NB
            for j in range(_NB):
                g_wait(j)
                w_start(c0 + j, j)
            for j in range(_NB):
                w_wait(j)
                g_start(c0 + _NB + j, j)
            return carry

        lax.fori_loop(0, ngroups - 1, body, 0)
        c0 = (ngroups - 1) * _NB
        for j in range(_NB):
            g_wait(j)
            w_start(c0 + j, j)
        for j in range(_NB):
            w_wait(j)

    return k(table, idx)


_COS_COEF = (0.9999999889445765, -19.739204466576158, 64.93911592834692,
             -85.45011342750767, 60.16742979194585, -25.966884612550658,
             6.527705962902734)


def _fast_cos(x):
    y = x * jnp.float32(1.0 / (2.0 * np.pi))
    big = jnp.float32(12582912.0)
    r = (y + big) - big
    y = y - r
    z = y * y
    acc = jnp.float32(_COS_COEF[-1])
    for c in _COS_COEF[-2::-1]:
        acc = acc * z + jnp.float32(c)
    return acc


def _layer_block(combine, bn, *refs):
    if combine:
        (nf_r, nbrf_r, ef_r, dt_r, mk_r, nbrz_r, selz_r, sels_r,
         wqa_r, qc_r, wk_r, bk_r, wv_r, bv_r,
         wo_r, w1a_r, w1b_r, b1_r, w2_r, b2_r, twp_r, tbp_r, out_r) = refs
    else:
        (nf_r, nbrf_r, ef_r, dt_r, mk_r,
         wqa_r, qc_r, wk_r, bk_r, wv_r, bv_r,
         wo_r, w1a_r, w1b_r, b1_r, w2_r, b2_r, twp_r, tbp_r, out_r) = refs

    nf = nf_r[...]
    nbr = nbrf_r[...]
    if combine:
        nbr = selz_r[...] * nbrz_r[...] + sels_r[...] * nbr
    tfx = _fast_cos(dt_r[...] * twp_r[...] + tbp_r[...])
    lane = lax.broadcasted_iota(jnp.int32, (1, _EMBED), 1)
    ef_pad = jnp.pad(ef_r[...], ((0, 0), (0, _EMBED - _EDGE)))
    eftf = jnp.where(lane < _EDGE, ef_pad, tfx)
    kin = jnp.concatenate([nbr, eftf], axis=1)

    kmat = jnp.dot(kin, wk_r[...], preferred_element_type=jnp.float32) + bk_r[...]
    vmat = jnp.dot(kin, wv_r[...], preferred_element_type=jnp.float32) + bv_r[...]
    q = jnp.dot(nf, wqa_r[...], preferred_element_type=jnp.float32) + qc_r[...]

    k3 = kmat.reshape(bn, _K, _QDIM)
    v3 = vmat.reshape(bn, _K, _QDIM)
    prod = q[:, None, :] * k3
    didx = lax.broadcasted_iota(jnp.int32, (1, 1, _QDIM), 2)
    head0 = didx < _DH
    s_all = jnp.sum(prod, axis=2)
    s0 = jnp.sum(jnp.where(head0, prod, 0.0), axis=2)
    s1 = s_all - s0
    scale = jnp.float32(1.0 / np.sqrt(_DH))
    m = mk_r[...] > 0.0
    s0 = jnp.where(m, s0 * scale, jnp.float32(-1e30))
    s1 = jnp.where(m, s1 * scale, jnp.float32(-1e30))
    e0 = jnp.exp(s0 - jnp.max(s0, axis=1, keepdims=True))
    e1 = jnp.exp(s1 - jnp.max(s1, axis=1, keepdims=True))
    a0 = e0 / jnp.sum(e0, axis=1, keepdims=True)
    a1 = e1 / jnp.sum(e1, axis=1, keepdims=True)
    attn = jnp.where(head0, a0[:, :, None], a1[:, :, None])
    outf = jnp.sum(attn * v3, axis=1)

    h = jnp.dot(outf, wo_r[...], preferred_element_type=jnp.float32)
    x = jnp.maximum(
        jnp.dot(h, w1a_r[...], preferred_element_type=jnp.float32)
        + jnp.dot(nf, w1b_r[...], preferred_element_type=jnp.float32)
        + b1_r[...], 0.0)
    out_r[...] = jnp.dot(x, w2_r[...], preferred_element_type=jnp.float32) + b2_r[...]


def _tc_layer(nf, nbrf, ef, dt, maskf, nbrz, selz, sels, weights, combine):
    n = nf.shape[0]
    bn = 256
    nb = n // bn
    bnk = bn * _K

    def rep(shape):
        return pl.BlockSpec(shape, lambda i: (0, 0))

    in_specs = [
        pl.BlockSpec((bn, _EMBED), lambda i: (i, 0)),
        pl.BlockSpec((bnk, _EMBED), lambda i: (i, 0)),
        pl.BlockSpec((bnk, _EDGE), lambda i: (i, 0)),
        pl.BlockSpec((bnk, 1), lambda i: (i, 0)),
        pl.BlockSpec((bn, _K), lambda i: (i, 0)),
    ]
    args = [nf, nbrf, ef, dt, maskf]
    if combine:
        in_specs += [
            pl.BlockSpec((bnk, _EMBED), lambda i: (i, 0)),
            pl.BlockSpec((bnk, 1), lambda i: (i, 0)),
            pl.BlockSpec((bnk, 1), lambda i: (i, 0)),
        ]
        args += [nbrz, selz, sels]
    in_specs += [rep(w.shape) for w in weights]
    args += list(weights)

    return pl.pallas_call(
        functools.partial(_layer_block, combine, bn),
        grid=(nb,),
        in_specs=in_specs,
        out_specs=pl.BlockSpec((bn, _EMBED), lambda i: (i, 0)),
        out_shape=jax.ShapeDtypeStruct((n, _EMBED), jnp.float32),
    )(*args)


def _prep_weights(Wq, bq, Wk, bk, Wv, bv, Wo, bo, W1, b1, W2, b2, tw, tb):
    qc = (jnp.cos(tb)[None, :] @ Wq[:, _EMBED:].T + bq[None, :])
    wqa = Wq[:, :_EMBED].T
    wk = jnp.pad(Wk.T, ((0, 12), (0, 0)))
    wv = jnp.pad(Wv.T, ((0, 12), (0, 0)))
    wo = Wo.T
    w1a = W1[:, :_QDIM].T
    w1b = W1[:, _QDIM:].T
    b1e = (b1 + bo @ W1[:, :_QDIM].T)[None, :]
    w2 = W2.T
    zeros16 = jnp.zeros((_EDGE,), jnp.float32)
    zeros12 = jnp.zeros((12,), jnp.float32)
    twp = jnp.concatenate([zeros16, tw, zeros12])[None, :]
    tbp = jnp.concatenate([zeros16, tb, zeros12])[None, :]
    return (wqa, qc, wk, bk[None, :], wv, bv[None, :],
            wo, w1a, w1b, b1e, w2, b2[None, :], twp, tbp)


def kernel(static_node_feat, nids0, nids1, nbr_nids0, nbr_nids1, nbr_mask0,
           nbr_mask1, times0, times1, nbr_times0, nbr_times1, nbr_feats0,
           nbr_feats1, g2l, tw, tb,
           Wq0, bq0, Wk0, bk0, Wv0, bv0, Wo0, bo0, W10, b10, W20, b20,
           Wq1, bq1, Wk1, bk1, Wv1, bv1, Wo1, bo1, W11, b11, W21, b21):
    f32 = jnp.float32
    i32 = jnp.int32

    remap_idx = jnp.concatenate(
        [nids1, nbr_nids0.reshape(-1), nids0])
    remap = _sc_elem_gather(g2l, remap_idx)
    l1 = remap[:_B1]
    lnbr0 = remap[_B1:_B1 + _B1]
    l0 = remap[_B1 + _B1:]

    feat_idx0 = jnp.concatenate([
        nids0, nbr_nids0.reshape(-1),
        jnp.arange(14336, dtype=i32),
    ])
    rows0 = _sc_row_gather(static_node_feat, feat_idx0)
    nf0 = rows0[:_B0]
    nbrf0 = rows0[_B0:_B0 + _B0 * _K]

    p1 = _prep_weights(Wq1, bq1, Wk1, bk1, Wv1, bv1, Wo1, bo1, W11, b11,
                       W21, b21, tw, tb)
    m1f = nbr_mask1.astype(f32)
    dt1 = jnp.where(nbr_mask1, times1[:, None] - nbr_times1, 0.0)
    half = _B1 // 2
    out1_parts = []
    for ci in range(2):
        sl = slice(ci * half, (ci + 1) * half)
        fidx = jnp.concatenate([nids1[sl], nbr_nids1[sl].reshape(-1)])
        rows1 = _sc_row_gather(static_node_feat, fidx)
        out1_parts.append(_tc_layer(
            rows1[:half], rows1[half:],
            nbr_feats1[sl].reshape(half * _K, _EDGE),
            dt1[sl].reshape(half * _K, 1), m1f[sl], None, None, None,
            p1, combine=False))
    out1 = jnp.concatenate(out1_parts)

    win1 = jnp.full((_NU,), -1, i32).at[l1].max(jnp.arange(_B1, dtype=i32))
    win0 = jnp.full((_NU,), -1, i32).at[l0].max(jnp.arange(_B0, dtype=i32))

    w = _sc_elem_gather(win1, lnbr0)
    valid = w >= 0
    zidx = jnp.where(valid, w, jnp.arange(_B1, dtype=i32))
    nbrz = _sc_row_gather(out1, zidx)
    m0flat = nbr_mask0.reshape(-1)
    selz = (m0flat & valid).astype(f32)[:, None]
    sels = (~m0flat).astype(f32)[:, None]

    p0 = _prep_weights(Wq0, bq0, Wk0, bk0, Wv0, bv0, Wo0, bo0, W10, b10,
                       W20, b20, tw, tb)
    m0f = nbr_mask0.astype(f32)
    dt0 = jnp.where(nbr_mask0, times0[:, None] - nbr_times0, 0.0)
    out0 = _tc_layer(nf0, nbrf0, nbr_feats0.reshape(_B0 * _K, _EDGE),
                     dt0.reshape(_B0 * _K, 1), m0f, nbrz, selz, sels,
                     p0, combine=True)

    table = jnp.concatenate(
        [out1, out0, jnp.zeros((32, _EMBED), f32)], axis=0)
    u = jnp.arange(_NU, dtype=i32)
    winf = jnp.where(win0 >= 0, _B1 + win0,
                     jnp.where(win1 >= 0, win1, _B1 + _B0 + (u % 32)))
    pad = _B1 + _B0 + (jnp.arange(49152 - _NU, dtype=i32) % 32)
    zrows = _sc_row_gather(table, jnp.concatenate([winf, pad]))
    return zrows[:_NU]


# speedup vs baseline: 1.4714x; 1.0103x over previous
import functools

import jax
import jax.numpy as jnp
import numpy as np
from jax import lax
from jax.experimental import pallas as pl
from jax.experimental.pallas import tpu as pltpu
from jax.experimental.pallas import tpu_sc as plsc

_EMBED = 128
_TIME = 100
_EDGE = 16
_QDIM = _EMBED + _TIME
_DH = _QDIM // 2
_NN = 100000
_B0 = 2048
_K = 16
_B1 = _B0 * _K
_NU = _B0 + _B1

_NC = 2
_NS = 16
_NW = _NC * _NS
_L = 16
_C = 128


def _wid():
    return lax.axis_index("s") * _NC + lax.axis_index("c")


def _sc_elem_gather(table, idx):
    n = idx.shape[0]
    t = table.shape[0]
    per_w = n // _NW
    nv = per_w // _L
    mesh = plsc.VectorSubcoreMesh(core_axis_name="c", subcore_axis_name="s")

    @functools.partial(
        pl.kernel,
        mesh=mesh,
        compiler_params=pltpu.CompilerParams(needs_layout_passes=False),
        out_type=jax.ShapeDtypeStruct((n,), jnp.int32),
        scratch_types=[
            pltpu.VMEM((t,), jnp.int32),
            pltpu.VMEM((per_w,), jnp.int32),
            pltpu.VMEM((per_w,), jnp.int32),
        ],
    )
    def k(table_hbm, idx_hbm, out_hbm, table_v, idx_v, out_v):
        w = _wid()
        base = w * per_w
        pltpu.sync_copy(table_hbm, table_v)
        pltpu.sync_copy(idx_hbm.at[pl.ds(base, per_w)], idx_v)

        def body(i, carry):
            iv = idx_v[pl.ds(i * _L, _L)]
            out_v[pl.ds(i * _L, _L)] = plsc.load_gather(table_v, [iv])
            return carry

        lax.fori_loop(0, nv, body, 0)
        pltpu.sync_copy(out_v, out_hbm.at[pl.ds(base, per_w)])

    return k(table, idx)


_NB = 4


def _sc_row_gather(table, idx):
    n = idx.shape[0]
    d = table.shape[1]
    per_w = n // _NW
    chunks = per_w // _C
    ngroups = chunks // _NB
    mesh = plsc.VectorSubcoreMesh(core_axis_name="c", subcore_axis_name="s")

    @functools.partial(
        pl.kernel,
        mesh=mesh,
        compiler_params=pltpu.CompilerParams(needs_layout_passes=False),
        cost_estimate=pl.CostEstimate(
            flops=0, transcendentals=0,
            bytes_accessed=2 * n * d * 4 + n * 4),
        out_type=jax.ShapeDtypeStruct((n, d), jnp.float32),
        scratch_types=[
            pltpu.VMEM((per_w,), jnp.int32),
        ] + [pltpu.VMEM((_C, d), jnp.float32)] * _NB
          + [pltpu.SemaphoreType.DMA] * (2 * _NB),
    )
    def k(table_hbm, idx_hbm, out_hbm, idx_v, *rest):
        bufs = rest[:_NB]
        gs = rest[_NB:2 * _NB]
        ws = rest[2 * _NB:]
        w = _wid()
        base = w * per_w
        pltpu.sync_copy(idx_hbm.at[pl.ds(base, per_w)], idx_v)

        def g_start(c, j):
            pltpu.async_copy(
                table_hbm.at[idx_v.at[pl.ds(c * _C, _C)]], bufs[j], gs[j])

        def g_wait(j):
            pltpu.make_async_copy(
                table_hbm.at[idx_v.at[pl.ds(0, _C)]], bufs[j], gs[j]).wait()

        def w_start(c, j):
            pltpu.async_copy(bufs[j], out_hbm.at[pl.ds(base + c * _C, _C)],
                             ws[j])

        def w_wait(j):
            pltpu.make_async_copy(bufs[j], out_hbm.at[pl.ds(base, _C)],
                                  ws[j]).wait()

        for j in range(_NB):
            g_start(j, j)

        def body(g, carry):
            c0 = g * _NB
            for j in range(_NB):
                g_wait(j)
                w_start(c0 + j, j)
            for j in range(_NB):
                w_wait(j)
                g_start(c0 + _NB + j, j)
            return carry

        lax.fori_loop(0, ngroups - 1, body, 0)
        c0 = (ngroups - 1) * _NB
        for j in range(_NB):
            g_wait(j)
            w_start(c0 + j, j)
        for j in range(_NB):
            w_wait(j)

    return k(table, idx)


_COS_COEF = (0.9999999889445765, -19.739204466576158, 64.93911592834692,
             -85.45011342750767, 60.16742979194585, -25.966884612550658,
             6.527705962902734)


def _fast_cos(x):
    y = x * jnp.float32(1.0 / (2.0 * np.pi))
    big = jnp.float32(12582912.0)
    r = (y + big) - big
    y = y - r
    z = y * y
    acc = jnp.float32(_COS_COEF[-1])
    for c in _COS_COEF[-2::-1]:
        acc = acc * z + jnp.float32(c)
    return acc


def _layer_block(combine, bn, *refs):
    if combine:
        (nf_r, nbrf_r, ef_r, dt_r, mk_r, nbrz_r, selz_r, sels_r,
         wqa_r, qc_r, wk_r, bk_r, wv_r, bv_r,
         wo_r, w1a_r, w1b_r, b1_r, w2_r, b2_r, twp_r, tbp_r, out_r) = refs
    else:
        (nf_r, nbrf_r, ef_r, dt_r, mk_r,
         wqa_r, qc_r, wk_r, bk_r, wv_r, bv_r,
         wo_r, w1a_r, w1b_r, b1_r, w2_r, b2_r, twp_r, tbp_r, out_r) = refs

    nf = nf_r[...]
    nbr = nbrf_r[...]
    if combine:
        nbr = selz_r[...] * nbrz_r[...] + sels_r[...] * nbr
    tfx = _fast_cos(dt_r[...] * twp_r[...] + tbp_r[...])
    lane = lax.broadcasted_iota(jnp.int32, (1, _EMBED), 1)
    ef_pad = jnp.pad(ef_r[...], ((0, 0), (0, _EMBED - _EDGE)))
    eftf = jnp.where(lane < _EDGE, ef_pad, tfx)
    kin = jnp.concatenate([nbr, eftf], axis=1)

    kmat = jnp.dot(kin, wk_r[...], preferred_element_type=jnp.float32) + bk_r[...]
    vmat = jnp.dot(kin, wv_r[...], preferred_element_type=jnp.float32) + bv_r[...]
    q = jnp.dot(nf, wqa_r[...], preferred_element_type=jnp.float32) + qc_r[...]

    k3 = kmat.reshape(bn, _K, _QDIM)
    v3 = vmat.reshape(bn, _K, _QDIM)
    prod = q[:, None, :] * k3
    didx = lax.broadcasted_iota(jnp.int32, (1, 1, _QDIM), 2)
    head0 = didx < _DH
    s_all = jnp.sum(prod, axis=2)
    s0 = jnp.sum(jnp.where(head0, prod, 0.0), axis=2)
    s1 = s_all - s0
    scale = jnp.float32(1.0 / np.sqrt(_DH))
    m = mk_r[...] > 0.0
    s0 = jnp.where(m, s0 * scale, jnp.float32(-1e30))
    s1 = jnp.where(m, s1 * scale, jnp.float32(-1e30))
    e0 = jnp.exp(s0 - jnp.max(s0, axis=1, keepdims=True))
    e1 = jnp.exp(s1 - jnp.max(s1, axis=1, keepdims=True))
    a0 = e0 / jnp.sum(e0, axis=1, keepdims=True)
    a1 = e1 / jnp.sum(e1, axis=1, keepdims=True)
    attn = jnp.where(head0, a0[:, :, None], a1[:, :, None])
    outf = jnp.sum(attn * v3, axis=1)

    h = jnp.dot(outf, wo_r[...], preferred_element_type=jnp.float32)
    x = jnp.maximum(
        jnp.dot(h, w1a_r[...], preferred_element_type=jnp.float32)
        + jnp.dot(nf, w1b_r[...], preferred_element_type=jnp.float32)
        + b1_r[...], 0.0)
    out_r[...] = jnp.dot(x, w2_r[...], preferred_element_type=jnp.float32) + b2_r[...]


def _tc_layer(nf, nbrf, ef, dt, maskf, nbrz, selz, sels, weights, combine):
    n = nf.shape[0]
    bn = 256
    nb = n // bn
    bnk = bn * _K

    def rep(shape):
        return pl.BlockSpec(shape, lambda i: (0, 0))

    in_specs = [
        pl.BlockSpec((bn, _EMBED), lambda i: (i, 0)),
        pl.BlockSpec((bnk, _EMBED), lambda i: (i, 0)),
        pl.BlockSpec((bnk, _EDGE), lambda i: (i, 0)),
        pl.BlockSpec((bnk, 1), lambda i: (i, 0)),
        pl.BlockSpec((bn, _K), lambda i: (i, 0)),
    ]
    args = [nf, nbrf, ef, dt, maskf]
    if combine:
        in_specs += [
            pl.BlockSpec((bnk, _EMBED), lambda i: (i, 0)),
            pl.BlockSpec((bnk, 1), lambda i: (i, 0)),
            pl.BlockSpec((bnk, 1), lambda i: (i, 0)),
        ]
        args += [nbrz, selz, sels]
    in_specs += [rep(w.shape) for w in weights]
    args += list(weights)

    return pl.pallas_call(
        functools.partial(_layer_block, combine, bn),
        grid=(nb,),
        in_specs=in_specs,
        out_specs=pl.BlockSpec((bn, _EMBED), lambda i: (i, 0)),
        out_shape=jax.ShapeDtypeStruct((n, _EMBED), jnp.float32),
    )(*args)


def _prep_weights(Wq, bq, Wk, bk, Wv, bv, Wo, bo, W1, b1, W2, b2, tw, tb):
    qc = (jnp.cos(tb)[None, :] @ Wq[:, _EMBED:].T + bq[None, :])
    wqa = Wq[:, :_EMBED].T
    wk = jnp.pad(Wk.T, ((0, 12), (0, 0)))
    wv = jnp.pad(Wv.T, ((0, 12), (0, 0)))
    wo = Wo.T
    w1a = W1[:, :_QDIM].T
    w1b = W1[:, _QDIM:].T
    b1e = (b1 + bo @ W1[:, :_QDIM].T)[None, :]
    w2 = W2.T
    zeros16 = jnp.zeros((_EDGE,), jnp.float32)
    zeros12 = jnp.zeros((12,), jnp.float32)
    twp = jnp.concatenate([zeros16, tw, zeros12])[None, :]
    tbp = jnp.concatenate([zeros16, tb, zeros12])[None, :]
    return (wqa, qc, wk, bk[None, :], wv, bv[None, :],
            wo, w1a, w1b, b1e, w2, b2[None, :], twp, tbp)


def kernel(static_node_feat, nids0, nids1, nbr_nids0, nbr_nids1, nbr_mask0,
           nbr_mask1, times0, times1, nbr_times0, nbr_times1, nbr_feats0,
           nbr_feats1, g2l, tw, tb,
           Wq0, bq0, Wk0, bk0, Wv0, bv0, Wo0, bo0, W10, b10, W20, b20,
           Wq1, bq1, Wk1, bk1, Wv1, bv1, Wo1, bo1, W11, b11, W21, b21):
    f32 = jnp.float32
    i32 = jnp.int32

    remap_idx = jnp.concatenate(
        [nids1, nbr_nids0.reshape(-1), nids0])
    remap = _sc_elem_gather(g2l, remap_idx)
    l1 = remap[:_B1]
    lnbr0 = remap[_B1:_B1 + _B1]
    l0 = remap[_B1 + _B1:]

    feat_idx0 = jnp.concatenate([
        nids0, nbr_nids0.reshape(-1),
        jnp.arange(14336, dtype=i32),
    ])
    rows0 = _sc_row_gather(static_node_feat, feat_idx0)
    nf0 = rows0[:_B0]
    nbrf0 = rows0[_B0:_B0 + _B0 * _K]

    p1 = _prep_weights(Wq1, bq1, Wk1, bk1, Wv1, bv1, Wo1, bo1, W11, b11,
                       W21, b21, tw, tb)
    m1f = nbr_mask1.astype(f32)
    dt1 = jnp.where(nbr_mask1, times1[:, None] - nbr_times1, 0.0)
    half = _B1 // 2
    out1_parts = []
    for ci in range(2):
        sl = slice(ci * half, (ci + 1) * half)
        fidx = jnp.concatenate([nids1[sl], nbr_nids1[sl].reshape(-1)])
        rows1 = _sc_row_gather(static_node_feat, fidx)
        out1_parts.append(_tc_layer(
            rows1[:half], rows1[half:],
            nbr_feats1[sl].reshape(half * _K, _EDGE),
            dt1[sl].reshape(half * _K, 1), m1f[sl], None, None, None,
            p1, combine=False))
    out1 = jnp.concatenate(out1_parts)

    win1 = jnp.full((_NU,), -1, i32).at[l1].max(jnp.arange(_B1, dtype=i32))
    win0 = jnp.full((_NU,), -1, i32).at[l0].max(jnp.arange(_B0, dtype=i32))

    w = _sc_elem_gather(win1, lnbr0)
    valid = w >= 0
    zidx = jnp.where(valid, w, jnp.arange(_B1, dtype=i32))
    nbrz = _sc_row_gather(out1, zidx)
    m0flat = nbr_mask0.reshape(-1)
    selz = (m0flat & valid).astype(f32)[:, None]
    sels = (~m0flat).astype(f32)[:, None]

    p0 = _prep_weights(Wq0, bq0, Wk0, bk0, Wv0, bv0, Wo0, bo0, W10, b10,
                       W20, b20, tw, tb)
    m0f = nbr_mask0.astype(f32)
    dt0 = jnp.where(nbr_mask0, times0[:, None] - nbr_times0, 0.0)
    out0 = _tc_layer(nf0, nbrf0, nbr_feats0.reshape(_B0 * _K, _EDGE),
                     dt0.reshape(_B0 * _K, 1), m0f, nbrz, selz, sels,
                     p0, combine=True)

    table = jnp.concatenate(
        [out1, out0, jnp.zeros((32, _EMBED), f32)], axis=0)
    u = jnp.arange(_NU, dtype=i32)
    winf = jnp.where(win0 >= 0, _B1 + win0,
                     jnp.where(win1 >= 0, win1, _B1 + _B0 + (u % 32)))
    pad = _B1 + _B0 + (jnp.arange(49152 - _NU, dtype=i32) % 32)
    zrows = _sc_row_gather(table, jnp.concatenate([winf, pad]))
    return zrows[:_NU]


# speedup vs baseline: 1.5069x; 1.0241x over previous
import functools

import jax
import jax.numpy as jnp
import numpy as np
from jax import lax
from jax.experimental import pallas as pl
from jax.experimental.pallas import tpu as pltpu
from jax.experimental.pallas import tpu_sc as plsc

_EMBED = 128
_TIME = 100
_EDGE = 16
_QDIM = _EMBED + _TIME
_DH = _QDIM // 2
_NN = 100000
_B0 = 2048
_K = 16
_B1 = _B0 * _K
_NU = _B0 + _B1

_NC = 2
_NS = 16
_NW = _NC * _NS
_L = 16
_C = 128


def _wid():
    return lax.axis_index("s") * _NC + lax.axis_index("c")


def _sc_elem_gather(table, idx):
    n = idx.shape[0]
    t = table.shape[0]
    per_w = n // _NW
    nv = per_w // _L
    mesh = plsc.VectorSubcoreMesh(core_axis_name="c", subcore_axis_name="s")

    @functools.partial(
        pl.kernel,
        mesh=mesh,
        compiler_params=pltpu.CompilerParams(needs_layout_passes=False),
        out_type=jax.ShapeDtypeStruct((n,), jnp.int32),
        scratch_types=[
            pltpu.VMEM((t,), jnp.int32),
            pltpu.VMEM((per_w,), jnp.int32),
            pltpu.VMEM((per_w,), jnp.int32),
        ],
    )
    def k(table_hbm, idx_hbm, out_hbm, table_v, idx_v, out_v):
        w = _wid()
        base = w * per_w
        pltpu.sync_copy(table_hbm, table_v)
        pltpu.sync_copy(idx_hbm.at[pl.ds(base, per_w)], idx_v)

        def body(i, carry):
            iv = idx_v[pl.ds(i * _L, _L)]
            out_v[pl.ds(i * _L, _L)] = plsc.load_gather(table_v, [iv])
            return carry

        lax.fori_loop(0, nv, body, 0)
        pltpu.sync_copy(out_v, out_hbm.at[pl.ds(base, per_w)])

    return k(table, idx)


_NB = 4


def _sc_row_gather(table, idx, nb=_NB):
    n = idx.shape[0]
    d = table.shape[1]
    per_w = n // _NW
    chunks = per_w // _C
    ngroups = chunks // nb
    mesh = plsc.VectorSubcoreMesh(core_axis_name="c", subcore_axis_name="s")

    @functools.partial(
        pl.kernel,
        mesh=mesh,
        compiler_params=pltpu.CompilerParams(needs_layout_passes=False),
        cost_estimate=pl.CostEstimate(
            flops=0, transcendentals=0,
            bytes_accessed=2 * n * d * 4 + n * 4),
        out_type=jax.ShapeDtypeStruct((n, d), jnp.float32),
        scratch_types=[
            pltpu.VMEM((per_w,), jnp.int32),
        ] + [pltpu.VMEM((_C, d), jnp.float32)] * nb
          + [pltpu.SemaphoreType.DMA] * (2 * nb),
    )
    def k(table_hbm, idx_hbm, out_hbm, idx_v, *rest):
        bufs = rest[:nb]
        gs = rest[nb:2 * nb]
        ws = rest[2 * nb:]
        w = _wid()
        base = w * per_w
        pltpu.sync_copy(idx_hbm.at[pl.ds(base, per_w)], idx_v)

        def g_start(c, j):
            pltpu.async_copy(
                table_hbm.at[idx_v.at[pl.ds(c * _C, _C)]], bufs[j], gs[j])

        def g_wait(j):
            pltpu.make_async_copy(
                table_hbm.at[idx_v.at[pl.ds(0, _C)]], bufs[j], gs[j]).wait()

        def w_start(c, j):
            pltpu.async_copy(bufs[j], out_hbm.at[pl.ds(base + c * _C, _C)],
                             ws[j])

        def w_wait(j):
            pltpu.make_async_copy(bufs[j], out_hbm.at[pl.ds(base, _C)],
                                  ws[j]).wait()

        for j in range(nb):
            g_start(j, j)

        def body(g, carry):
            c0 = g * nb
            for j in range(nb):
                g_wait(j)
                w_start(c0 + j, j)
            for j in range(nb):
                w_wait(j)
                g_start(c0 + nb + j, j)
            return carry

        lax.fori_loop(0, ngroups - 1, body, 0)
        c0 = (ngroups - 1) * nb
        for j in range(nb):
            g_wait(j)
            w_start(c0 + j, j)
        for j in range(nb):
            w_wait(j)

    return k(table, idx)


_COS_COEF = (0.9999999889445765, -19.739204466576158, 64.93911592834692,
             -85.45011342750767, 60.16742979194585, -25.966884612550658,
             6.527705962902734)


def _fast_cos(x):
    y = x * jnp.float32(1.0 / (2.0 * np.pi))
    big = jnp.float32(12582912.0)
    r = (y + big) - big
    y = y - r
    z = y * y
    acc = jnp.float32(_COS_COEF[-1])
    for c in _COS_COEF[-2::-1]:
        acc = acc * z + jnp.float32(c)
    return acc


def _layer_block(combine, bn, *refs):
    if combine:
        (tbl_r, nf_r, nbrf_r, ef_r, dt_r, mk_r, nbrz_r, selz_r, sels_r,
         wqa_r, qc_r, wk_r, bk_r, wv_r, bv_r,
         wo_r, w1a_r, w1b_r, b1_r, w2_r, b2_r, twp_r, tbp_r, out_r) = refs
    else:
        (tbl_r, nf_r, nbrf_r, ef_r, dt_r, mk_r,
         wqa_r, qc_r, wk_r, bk_r, wv_r, bv_r,
         wo_r, w1a_r, w1b_r, b1_r, w2_r, b2_r, twp_r, tbp_r, out_r) = refs
    del tbl_r

    nf = nf_r[...]
    nbr = nbrf_r[...]
    if combine:
        nbr = selz_r[...] * nbrz_r[...] + sels_r[...] * nbr
    tfx = _fast_cos(dt_r[...] * twp_r[...] + tbp_r[...])
    lane = lax.broadcasted_iota(jnp.int32, (1, _EMBED), 1)
    ef_pad = jnp.pad(ef_r[...], ((0, 0), (0, _EMBED - _EDGE)))
    eftf = jnp.where(lane < _EDGE, ef_pad, tfx)
    kin = jnp.concatenate([nbr, eftf], axis=1)

    kmat = jnp.dot(kin, wk_r[...], preferred_element_type=jnp.float32) + bk_r[...]
    vmat = jnp.dot(kin, wv_r[...], preferred_element_type=jnp.float32) + bv_r[...]
    q = jnp.dot(nf, wqa_r[...], preferred_element_type=jnp.float32) + qc_r[...]

    k3 = kmat.reshape(bn, _K, _QDIM)
    v3 = vmat.reshape(bn, _K, _QDIM)
    prod = q[:, None, :] * k3
    didx = lax.broadcasted_iota(jnp.int32, (1, 1, _QDIM), 2)
    head0 = didx < _DH
    s_all = jnp.sum(prod, axis=2)
    s0 = jnp.sum(jnp.where(head0, prod, 0.0), axis=2)
    s1 = s_all - s0
    scale = jnp.float32(1.0 / np.sqrt(_DH))
    m = mk_r[...] > 0.0
    s0 = jnp.where(m, s0 * scale, jnp.float32(-1e30))
    s1 = jnp.where(m, s1 * scale, jnp.float32(-1e30))
    e0 = jnp.exp(s0 - jnp.max(s0, axis=1, keepdims=True))
    e1 = jnp.exp(s1 - jnp.max(s1, axis=1, keepdims=True))
    a0 = e0 / jnp.sum(e0, axis=1, keepdims=True)
    a1 = e1 / jnp.sum(e1, axis=1, keepdims=True)
    attn = jnp.where(head0, a0[:, :, None], a1[:, :, None])
    outf = jnp.sum(attn * v3, axis=1)

    h = jnp.dot(outf, wo_r[...], preferred_element_type=jnp.float32)
    x = jnp.maximum(
        jnp.dot(h, w1a_r[...], preferred_element_type=jnp.float32)
        + jnp.dot(nf, w1b_r[...], preferred_element_type=jnp.float32)
        + b1_r[...], 0.0)
    out_r[...] = jnp.dot(x, w2_r[...], preferred_element_type=jnp.float32) + b2_r[...]


def _tc_layer(table, row_off, nf, nbrf, ef, dt, maskf, nbrz, selz, sels,
              weights, combine):
    n = nf.shape[0]
    bn = 256
    nb = n // bn
    bnk = bn * _K
    off = row_off // bn

    def rep(shape):
        return pl.BlockSpec(shape, lambda i: (0, 0))

    in_specs = [
        pl.BlockSpec((8, _EMBED), lambda i: (0, 0)),
        pl.BlockSpec((bn, _EMBED), lambda i: (i, 0)),
        pl.BlockSpec((bnk, _EMBED), lambda i: (i, 0)),
        pl.BlockSpec((bnk, _EDGE), lambda i: (i, 0)),
        pl.BlockSpec((bnk, 1), lambda i: (i, 0)),
        pl.BlockSpec((bn, _K), lambda i: (i, 0)),
    ]
    args = [table, nf, nbrf, ef, dt, maskf]
    if combine:
        in_specs += [
            pl.BlockSpec((bnk, _EMBED), lambda i: (i, 0)),
            pl.BlockSpec((bnk, 1), lambda i: (i, 0)),
            pl.BlockSpec((bnk, 1), lambda i: (i, 0)),
        ]
        args += [nbrz, selz, sels]
    in_specs += [rep(w.shape) for w in weights]
    args += list(weights)

    return pl.pallas_call(
        functools.partial(_layer_block, combine, bn),
        grid=(nb,),
        in_specs=in_specs,
        out_specs=pl.BlockSpec((bn, _EMBED), lambda i: (i + off, 0)),
        out_shape=jax.ShapeDtypeStruct(table.shape, jnp.float32),
        input_output_aliases={0: 0},
    )(*args)


def _prep_weights(Wq, bq, Wk, bk, Wv, bv, Wo, bo, W1, b1, W2, b2, tw, tb):
    qc = (jnp.cos(tb)[None, :] @ Wq[:, _EMBED:].T + bq[None, :])
    wqa = Wq[:, :_EMBED].T
    wk = jnp.pad(Wk.T, ((0, 12), (0, 0)))
    wv = jnp.pad(Wv.T, ((0, 12), (0, 0)))
    wo = Wo.T
    w1a = W1[:, :_QDIM].T
    w1b = W1[:, _QDIM:].T
    b1e = (b1 + bo @ W1[:, :_QDIM].T)[None, :]
    w2 = W2.T
    zeros16 = jnp.zeros((_EDGE,), jnp.float32)
    zeros12 = jnp.zeros((12,), jnp.float32)
    twp = jnp.concatenate([zeros16, tw, zeros12])[None, :]
    tbp = jnp.concatenate([zeros16, tb, zeros12])[None, :]
    return (wqa, qc, wk, bk[None, :], wv, bv[None, :],
            wo, w1a, w1b, b1e, w2, b2[None, :], twp, tbp)


def kernel(static_node_feat, nids0, nids1, nbr_nids0, nbr_nids1, nbr_mask0,
           nbr_mask1, times0, times1, nbr_times0, nbr_times1, nbr_feats0,
           nbr_feats1, g2l, tw, tb,
           Wq0, bq0, Wk0, bk0, Wv0, bv0, Wo0, bo0, W10, b10, W20, b20,
           Wq1, bq1, Wk1, bk1, Wv1, bv1, Wo1, bo1, W11, b11, W21, b21):
    f32 = jnp.float32
    i32 = jnp.int32

    remap_idx = jnp.concatenate(
        [nids1, nbr_nids0.reshape(-1), nids0])
    remap = _sc_elem_gather(g2l, remap_idx)
    l1 = remap[:_B1]
    lnbr0 = remap[_B1:_B1 + _B1]
    l0 = remap[_B1 + _B1:]

    feat_idx0 = jnp.concatenate([
        nids0, nbr_nids0.reshape(-1),
        jnp.arange(2048, dtype=i32),
    ])
    rows0 = _sc_row_gather(static_node_feat, feat_idx0, nb=3)
    nf0 = rows0[:_B0]
    nbrf0 = rows0[_B0:_B0 + _B0 * _K]

    p1 = _prep_weights(Wq1, bq1, Wk1, bk1, Wv1, bv1, Wo1, bo1, W11, b11,
                       W21, b21, tw, tb)
    m1f = nbr_mask1.astype(f32)
    dt1 = jnp.where(nbr_mask1, times1[:, None] - nbr_times1, 0.0)
    half = _B1 // 2
    table = jnp.zeros((_B1 + _B0 + 32, _EMBED), f32)
    for ci in range(2):
        sl = slice(ci * half, (ci + 1) * half)
        fidx = jnp.concatenate([nids1[sl], nbr_nids1[sl].reshape(-1)])
        rows1 = _sc_row_gather(static_node_feat, fidx)
        table = _tc_layer(
            table, ci * half, rows1[:half], rows1[half:],
            nbr_feats1[sl].reshape(half * _K, _EDGE),
            dt1[sl].reshape(half * _K, 1), m1f[sl], None, None, None,
            p1, combine=False)

    win1 = jnp.full((_NU,), -1, i32).at[l1].max(jnp.arange(_B1, dtype=i32))
    win0 = jnp.full((_NU,), -1, i32).at[l0].max(jnp.arange(_B0, dtype=i32))

    w = _sc_elem_gather(win1, lnbr0)
    valid = w >= 0
    zidx = jnp.where(valid, w, jnp.arange(_B1, dtype=i32))
    nbrz = _sc_row_gather(table, zidx)
    m0flat = nbr_mask0.reshape(-1)
    selz = (m0flat & valid).astype(f32)[:, None]
    sels = (~m0flat).astype(f32)[:, None]

    p0 = _prep_weights(Wq0, bq0, Wk0, bk0, Wv0, bv0, Wo0, bo0, W10, b10,
                       W20, b20, tw, tb)
    m0f = nbr_mask0.astype(f32)
    dt0 = jnp.where(nbr_mask0, times0[:, None] - nbr_times0, 0.0)
    table = _tc_layer(table, _B1, nf0, nbrf0,
                      nbr_feats0.reshape(_B0 * _K, _EDGE),
                      dt0.reshape(_B0 * _K, 1), m0f, nbrz, selz, sels,
                      p0, combine=True)

    u = jnp.arange(_NU, dtype=i32)
    winf = jnp.where(win0 >= 0, _B1 + win0,
                     jnp.where(win1 >= 0, win1, _B1 + _B0 + (u % 32)))
    pad = _B1 + _B0 + (jnp.arange(40960 - _NU, dtype=i32) % 32)
    zrows = _sc_row_gather(table, jnp.concatenate([winf, pad]), nb=2)
    return zrows[:_NU]
